# baseline (speedup 1.0000x reference)
import functools

import jax
import jax.numpy as jnp
from jax import lax
from jax.experimental import pallas as pl
from jax.experimental.pallas import tpu as pltpu
from jax.experimental.pallas import tpu_sc as plsc

_NUM_OBJS = 151
_NUM_RELS = 51
_BATCH = 16384
_NROWS = _NUM_OBJS * _NUM_OBJS

_NC = 2
_NS = 16
_NW = _NC * _NS
_L = 16


def _gather_column(row_v, idx_v, col_v):
    def chunk(i, _):
        v = plsc.load_gather(row_v, [idx_v[pl.ds(i * _L, _L)]])
        col_v[pl.ds(i * _L, _L)] = v
        return _

    lax.fori_loop(0, _BATCH // _L, chunk, 0, unroll=8)


def _freq_bias_body(idx_hbm, tableT_hbm, outT_hbm,
                    idx_v, row0_v, row1_v, col0_v, col1_v, sem):
    wid = lax.axis_index("s") * _NC + lax.axis_index("c")

    cp_idx = pltpu.async_copy(idx_hbm, idx_v, sem)
    cp_r0 = pltpu.async_copy(tableT_hbm.at[wid], row0_v, sem)
    second = wid + _NW < _NUM_RELS

    @pl.when(second)
    def _():
        pltpu.async_copy(tableT_hbm.at[wid + _NW], row1_v, sem).wait()

    cp_idx.wait()
    cp_r0.wait()

    _gather_column(row0_v, idx_v, col0_v)
    pltpu.sync_copy(col0_v, outT_hbm.at[wid])

    @pl.when(second)
    def _():
        _gather_column(row1_v, idx_v, col1_v)
        pltpu.sync_copy(col1_v, outT_hbm.at[wid + _NW])


_freq_bias = functools.partial(
    pl.kernel,
    out_type=jax.ShapeDtypeStruct((_NUM_RELS, _BATCH), jnp.float32),
    mesh=plsc.VectorSubcoreMesh(core_axis_name="c", subcore_axis_name="s"),
    compiler_params=pltpu.CompilerParams(use_tc_tiling_on_sc=True,
                                         needs_layout_passes=False),
    scratch_types=[
        pltpu.VMEM((_BATCH,), jnp.int32),
        pltpu.VMEM((_NROWS,), jnp.float32),
        pltpu.VMEM((_NROWS,), jnp.float32),
        pltpu.VMEM((_BATCH,), jnp.float32),
        pltpu.VMEM((_BATCH,), jnp.float32),
        pltpu.SemaphoreType.DMA,
    ],
)(_freq_bias_body)


def kernel(labels, obj_baseline_weight):
    labels = labels.astype(jnp.int32)
    pair_idx = labels[:, 0] * _NUM_OBJS + labels[:, 1]
    outT = _freq_bias(pair_idx, obj_baseline_weight.T)
    return outT.T


# speedup vs baseline: 1.5497x; 1.5497x over previous
import functools

import jax
import jax.numpy as jnp
from jax import lax
from jax.experimental import pallas as pl
from jax.experimental.pallas import tpu as pltpu
from jax.experimental.pallas import tpu_sc as plsc

_NUM_OBJS = 151
_NUM_RELS = 51
_BATCH = 16384
_NROWS = _NUM_OBJS * _NUM_OBJS

_NC = 2
_NS = 16
_NW = _NC * _NS
_L = 16


def _gather_column(row_v, idx_v, col_v):
    @plsc.parallel_loop(0, _BATCH, step=_L, unroll=8)
    def chunk(i):
        v = plsc.load_gather(row_v, [idx_v[pl.ds(i, _L)]])
        col_v[pl.ds(i, _L)] = v


def _freq_bias_body(idx_hbm, tableT_hbm, outT_hbm,
                    idx_v, row0_v, row1_v, col0_v, col1_v, sem):
    wid = lax.axis_index("s") * _NC + lax.axis_index("c")

    cp_idx = pltpu.async_copy(idx_hbm, idx_v, sem)
    cp_r0 = pltpu.async_copy(tableT_hbm.at[wid], row0_v, sem)
    second = wid + _NW < _NUM_RELS

    @pl.when(second)
    def _():
        pltpu.async_copy(tableT_hbm.at[wid + _NW], row1_v, sem).wait()

    cp_idx.wait()
    cp_r0.wait()

    _gather_column(row0_v, idx_v, col0_v)
    pltpu.sync_copy(col0_v, outT_hbm.at[wid])

    @pl.when(second)
    def _():
        _gather_column(row1_v, idx_v, col1_v)
        pltpu.sync_copy(col1_v, outT_hbm.at[wid + _NW])


_freq_bias = functools.partial(
    pl.kernel,
    out_type=jax.ShapeDtypeStruct((_NUM_RELS, _BATCH), jnp.float32),
    mesh=plsc.VectorSubcoreMesh(core_axis_name="c", subcore_axis_name="s"),
    compiler_params=pltpu.CompilerParams(use_tc_tiling_on_sc=True,
                                         needs_layout_passes=False),
    scratch_types=[
        pltpu.VMEM((_BATCH,), jnp.int32),
        pltpu.VMEM((_NROWS,), jnp.float32),
        pltpu.VMEM((_NROWS,), jnp.float32),
        pltpu.VMEM((_BATCH,), jnp.float32),
        pltpu.VMEM((_BATCH,), jnp.float32),
        pltpu.SemaphoreType.DMA,
    ],
)(_freq_bias_body)


def kernel(labels, obj_baseline_weight):
    labels = labels.astype(jnp.int32)
    pair_idx = labels[:, 0] * _NUM_OBJS + labels[:, 1]
    outT = _freq_bias(pair_idx, obj_baseline_weight.T)
    return outT.T
